# baseline (speedup 1.0000x reference)
import functools
import jax
import jax.numpy as jnp
from jax import lax
from jax.experimental import pallas as pl
from jax.experimental.pallas import tpu as pltpu
from jax.experimental.pallas import tpu_sc as plsc

N = 10000
E = 320000
D = 128
NG = 64
NC, NS, L = 2, 16, 16
NW = NC * NS
NP = 10240
RPW = NP // NW
CK = 64
NCHUNK = 160
EP = NW * NCHUNK * CK
DUMMY_SRC = N + 200
DUMMY_DST = NP - 1
EPT = EP // NW
ZR = 160

_mesh = plsc.VectorSubcoreMesh(
    core_axis_name="c", subcore_axis_name="s", num_cores=NC, num_subcores=NS)

_f32 = jnp.float32


def _wid(c, s):
    return c * NS + s


def _deg_body(dst_hbm, cnt_out, dstbuf, cntbuf, partials, redbuf, sumbuf):
    c = lax.axis_index("c")
    s = lax.axis_index("s")
    w = _wid(c, s)

    def zero(i, carry):
        cntbuf[pl.ds(i * L, L)] = jnp.zeros((L,), _f32)
        return carry
    lax.fori_loop(0, NP // L, zero, 0)

    pltpu.sync_copy(dst_hbm.at[pl.ds(w * NCHUNK, NCHUNK), :], dstbuf)
    ones = jnp.ones((L,), _f32)

    def count(r, carry):
        for j in range(CK // L):
            idx = dstbuf[r, pl.ds(j * L, L)]
            plsc.addupdate_scatter(cntbuf, [idx], ones)
        return carry
    lax.fori_loop(0, NCHUNK, count, 0)

    pltpu.sync_copy(cntbuf, partials.at[s])
    plsc.subcore_barrier()

    cols = NP // NS
    col0 = s * cols
    pltpu.sync_copy(partials.at[:, pl.ds(col0, cols)], redbuf)

    def red_col(j, carry):
        def red_row(t, acc):
            return acc + redbuf[t, pl.ds(j * L, L)]
        acc = lax.fori_loop(1, NS, red_row, redbuf[0, pl.ds(j * L, L)])
        sumbuf[pl.ds(j * L, L)] = acc
        return carry
    lax.fori_loop(0, cols // L, red_col, 0)

    pltpu.sync_copy(sumbuf, cnt_out.at[c, pl.ds(col0, cols)])


_deg_kernel = functools.partial(
    pl.kernel, _deg_body,
    out_type=jax.ShapeDtypeStruct((NC, NP), _f32),
    mesh=_mesh,
    compiler_params=pltpu.CompilerParams(needs_layout_passes=False),
    scratch_types=[
        pltpu.VMEM((NCHUNK, CK), jnp.int32),
        pltpu.VMEM((NP,), _f32),
        pltpu.VMEM_SHARED((NS, NP), _f32),
        pltpu.VMEM((NS, NP // NS), _f32),
        pltpu.VMEM((NP // NS,), _f32),
    ])()


def _agg_body(y_hbm, src_hbm, dst_hbm, parts_out, sbuf, dbuf, rbs, acc,
              sgs, sss):
    c = lax.axis_index("c")
    s = lax.axis_index("s")
    w = _wid(c, s)

    def zero_row(i, carry):
        for j in range(D // L):
            rbs[0][i, pl.ds(j * L, L)] = jnp.zeros((L,), _f32)
        return carry
    lax.fori_loop(0, CK, zero_row, 0)
    rpt = NP // NS
    for k in range(rpt // CK):
        pltpu.sync_copy(rbs[0], acc.at[pl.ds(s * rpt + k * CK, CK), :])
    plsc.subcore_barrier()

    def gather_start(g, b3):
        pltpu.async_copy(y_hbm.at[sbuf.at[g]], rbs[b3], sgs[b3])

    def gather_wait(g, b3):
        pltpu.make_async_copy(y_hbm.at[sbuf.at[g]], rbs[b3],
                              sgs[b3]).wait()

    def scatter_start(g, b3):
        pltpu.async_copy(rbs[b3], acc.at[dbuf.at[g]], sss[b3], add=True)

    def scatter_wait(g, b3):
        pltpu.make_async_copy(rbs[b3], acc.at[dbuf.at[g]],
                              sss[b3]).wait()

    PH = NCHUNK // 2
    for phase in range(2):
        pltpu.sync_copy(
            src_hbm.at[pl.ds(w * NCHUNK + phase * PH, PH), :], sbuf)
        pltpu.sync_copy(
            dst_hbm.at[pl.ds(w * NCHUNK + phase * PH, PH), :], dbuf)
        gather_start(0, 0)
        gather_start(1, 1)

        def sweep(t, carry):
            for u in range(3):
                g = 3 * t + u
                b3 = u

                @pl.when(g < PH)
                def _():
                    gather_wait(g, b3)
                    scatter_start(g, b3)

                @pl.when(jnp.logical_and(g >= 1, g + 2 < PH))
                def _():
                    scatter_wait(g - 1, (u + 2) % 3)

                @pl.when(g + 2 < PH)
                def _():
                    gather_start(g + 2, (u + 2) % 3)
            return carry
        lax.fori_loop(0, (PH + 2) // 3, sweep, 0)
        for g in (PH - 3, PH - 2, PH - 1):
            scatter_wait(g, g % 3)
    plsc.subcore_barrier()

    for k in range(rpt // CK):
        r0 = s * rpt + k * CK
        rb = rbs[k % 3]
        pltpu.sync_copy(acc.at[pl.ds(r0, CK), :], rb)
        pltpu.sync_copy(rb, parts_out.at[c, pl.ds(r0, CK), :])


_agg_kernel = functools.partial(
    pl.kernel, _agg_body,
    out_type=jax.ShapeDtypeStruct((NC, NP, D), _f32),
    mesh=_mesh,
    compiler_params=pltpu.CompilerParams(needs_layout_passes=False),
    scratch_types=[
        pltpu.VMEM((NCHUNK // 2, CK), jnp.int32),
        pltpu.VMEM((NCHUNK // 2, CK), jnp.int32),
        tuple(pltpu.VMEM((CK, D), _f32) for _ in range(3)),
        pltpu.VMEM_SHARED((NP, D), _f32),
        tuple(pltpu.SemaphoreType.DMA for _ in range(3)),
        tuple(pltpu.SemaphoreType.DMA for _ in range(3)),
    ])()


def _pool_body(h_hbm, batch_hbm, pooled_out, hbuf, bvmem, accv,
               partials, redbuf, outbuf):
    c = lax.axis_index("c")
    s = lax.axis_index("s")
    w = _wid(c, s)
    r0 = w * RPW

    neg = jnp.full((L,), -jnp.inf, _f32)

    def init(i, carry):
        accv[pl.ds(i * L, L)] = neg
        return carry
    lax.fori_loop(0, (NG + 1) * D // L, init, 0)

    pltpu.sync_copy(h_hbm.at[pl.ds(r0, RPW), :], hbuf)
    pltpu.sync_copy(batch_hbm.at[pl.ds(r0, RPW)], bvmem)

    lanes = lax.iota(jnp.int32, L)

    def row(r, carry):
        b_prev, run = carry
        b_r = plsc.load_gather(bvmem, [jnp.broadcast_to(r, (L,))])
        same = b_r == b_prev
        base = b_r * D
        new_run = []
        for j in range(D // L):
            v = hbuf[r, pl.ds(j * L, L)]
            m = jnp.where(same, jnp.maximum(run[j], v), v)
            plsc.store_scatter(accv, [base + (j * L) + lanes], m)
            new_run.append(m)
        return b_r, tuple(new_run)

    init_carry = (jnp.full((L,), -1, jnp.int32),
                  tuple(neg for _ in range(D // L)))
    lax.fori_loop(0, RPW, row, init_carry)

    pltpu.sync_copy(accv.at[pl.ds(0, NG * D)], partials.at[s])
    plsc.subcore_barrier()

    cols = NG * D // NS
    col0 = s * cols
    pltpu.sync_copy(partials.at[:, pl.ds(col0, cols)], redbuf)

    def red_col(j, carry):
        def red_row(t, acc):
            return jnp.maximum(acc, redbuf[t, pl.ds(j * L, L)])
        acc = lax.fori_loop(1, NS, red_row, redbuf[0, pl.ds(j * L, L)])
        outbuf[pl.ds(j * L, L)] = acc
        return carry
    lax.fori_loop(0, cols // L, red_col, 0)

    pltpu.sync_copy(outbuf, pooled_out.at[c, pl.ds(col0, cols)])


_pool_kernel = functools.partial(
    pl.kernel, _pool_body,
    out_type=jax.ShapeDtypeStruct((NC, NG * D), _f32),
    mesh=_mesh,
    compiler_params=pltpu.CompilerParams(needs_layout_passes=False),
    scratch_types=[
        pltpu.VMEM((RPW, D), _f32),
        pltpu.VMEM((RPW,), jnp.int32),
        pltpu.VMEM(((NG + 1) * D,), _f32),
        pltpu.VMEM_SHARED((NS, NG * D), _f32),
        pltpu.VMEM((NS, NG * D // NS), _f32),
        pltpu.VMEM((NG * D // NS,), _f32),
    ])()


_GRID = NP // RPW

_row_spec = pl.BlockSpec((RPW, D), lambda i: (i, 0))
_cnt_spec = pl.BlockSpec((RPW, 1), lambda i: (i, 0))
_full_spec = pl.BlockSpec((D, D), lambda i: (0, 0))
_bias_spec = pl.BlockSpec((1, D), lambda i: (0, 0))


def _scale_mm_body(c0_ref, c1_ref, x_ref, w_ref, o_ref):
    dinv = lax.rsqrt(1.0 + c0_ref[...] + c1_ref[...])
    o_ref[...] = jnp.dot(x_ref[...], w_ref[...],
                         preferred_element_type=_f32) * dinv


def _tc_scale_mm(c0, c1, x, w):
    return pl.pallas_call(
        _scale_mm_body,
        grid=(N // 400,),
        in_specs=[pl.BlockSpec((400, 1), lambda i: (i, 0)),
                  pl.BlockSpec((400, 1), lambda i: (i, 0)),
                  pl.BlockSpec((400, D), lambda i: (i, 0)),
                  _full_spec],
        out_specs=pl.BlockSpec((400, D), lambda i: (i, 0)),
        out_shape=jax.ShapeDtypeStruct((NP, D), _f32),
    )(c0, c1, x, w)


def _combine_mm_body(c0_ref, c1_ref, s0_ref, s1_ref, y_ref, b_ref, w_ref,
                     o_ref):
    dinv = lax.rsqrt(1.0 + c0_ref[...] + c1_ref[...])
    h = jnp.maximum(
        dinv * (s0_ref[...] + s1_ref[...] + y_ref[...]) + b_ref[...], 0.0)
    o_ref[...] = jnp.dot(h, w_ref[...], preferred_element_type=_f32) * dinv


def _tc_combine_mm(c0, c1, s0, s1, y, b, w):
    return pl.pallas_call(
        _combine_mm_body,
        grid=(_GRID,),
        in_specs=[_cnt_spec, _cnt_spec, _row_spec, _row_spec, _row_spec,
                  _bias_spec, _full_spec],
        out_specs=_row_spec,
        out_shape=jax.ShapeDtypeStruct((NP, D), _f32),
    )(c0, c1, s0, s1, y, b, w)


def _combine_body(c0_ref, c1_ref, s0_ref, s1_ref, y_ref, b_ref, o_ref):
    dinv = lax.rsqrt(1.0 + c0_ref[...] + c1_ref[...])
    o_ref[...] = jnp.maximum(
        dinv * (s0_ref[...] + s1_ref[...] + y_ref[...]) + b_ref[...], 0.0)


def _tc_combine(c0, c1, s0, s1, y, b):
    return pl.pallas_call(
        _combine_body,
        grid=(_GRID,),
        in_specs=[_cnt_spec, _cnt_spec, _row_spec, _row_spec, _row_spec,
                  _bias_spec],
        out_specs=_row_spec,
        out_shape=jax.ShapeDtypeStruct((NP, D), _f32),
    )(c0, c1, s0, s1, y, b)


def _head_body(p_ref, fw_ref, fb_ref, lw_ref, lb_ref, o_ref):
    pooled = jnp.max(p_ref[...], axis=0)
    z = jnp.maximum(
        jnp.dot(pooled, fw_ref[...], preferred_element_type=_f32)
        + fb_ref[...], 0.0)
    o_ref[...] = jnp.dot(z, lw_ref[...], preferred_element_type=_f32) \
        + lb_ref[...]


def _tc_head(pooled_parts, fc1_W, fc1_b, lin_W, lin_b):
    ncls = lin_W.shape[1]
    return pl.pallas_call(
        _head_body,
        in_specs=[
            pl.BlockSpec((NC, NG, D), lambda: (0, 0, 0)),
            pl.BlockSpec((D, D), lambda: (0, 0)),
            pl.BlockSpec((1, D), lambda: (0, 0)),
            pl.BlockSpec((D, ncls), lambda: (0, 0)),
            pl.BlockSpec((1, ncls), lambda: (0, 0)),
        ],
        out_specs=pl.BlockSpec((NG, ncls), lambda: (0, 0)),
        out_shape=jax.ShapeDtypeStruct((NG, ncls), _f32),
    )(pooled_parts, fc1_W, fc1_b, lin_W, lin_b)


@jax.jit
def kernel(x, edge_index, batch, W1, b1, W2, b2, W3, b3,
           fc1_W, fc1_b, lin_W, lin_b):
    spread = N + jnp.arange(EP - E, dtype=edge_index.dtype) % (NP - N)
    srcp = jnp.concatenate([edge_index[0], spread]).reshape(EP // CK, CK)
    dstp = jnp.concatenate([edge_index[1], spread]).reshape(EP // CK, CK)
    batchp = jnp.concatenate(
        [batch, jnp.full((NP - N,), NG, batch.dtype)], axis=0)

    cnt = _deg_kernel(dstp)
    c0 = cnt[0].reshape(NP, 1)
    c1 = cnt[1].reshape(NP, 1)

    y1 = _tc_scale_mm(c0, c1, x, W1)
    s1 = _agg_kernel(y1, srcp, dstp)
    y2 = _tc_combine_mm(c0, c1, s1[0], s1[1], y1, b1.reshape(1, D), W2)
    s2 = _agg_kernel(y2, srcp, dstp)
    y3 = _tc_combine_mm(c0, c1, s2[0], s2[1], y2, b2.reshape(1, D), W3)
    s3 = _agg_kernel(y3, srcp, dstp)
    h3 = _tc_combine(c0, c1, s3[0], s3[1], y3, b3.reshape(1, D))

    pooled = _pool_kernel(h3, batchp).reshape(NC, NG, D)
    return _tc_head(pooled, fc1_W, fc1_b.reshape(1, D),
                    lin_W, lin_b.reshape(1, lin_W.shape[1]))


# speedup vs baseline: 1.0395x; 1.0395x over previous
import functools
import jax
import jax.numpy as jnp
from jax import lax
from jax.experimental import pallas as pl
from jax.experimental.pallas import tpu as pltpu
from jax.experimental.pallas import tpu_sc as plsc

N = 10000
E = 320000
D = 128
NG = 64
NC, NS, L = 2, 16, 16
NW = NC * NS
NP = 10240
RPW = NP // NW
CK = 80
NCHUNK = 128
EP = NW * NCHUNK * CK
DUMMY_SRC = N + 200
DUMMY_DST = NP - 1
EPT = EP // NW
ZR = 160

_mesh = plsc.VectorSubcoreMesh(
    core_axis_name="c", subcore_axis_name="s", num_cores=NC, num_subcores=NS)

_f32 = jnp.float32


def _wid(c, s):
    return c * NS + s


def _deg_body(dst_hbm, cnt_out, dstbuf, cntbuf, partials, redbuf, sumbuf):
    c = lax.axis_index("c")
    s = lax.axis_index("s")
    w = _wid(c, s)

    def zero(i, carry):
        cntbuf[pl.ds(i * L, L)] = jnp.zeros((L,), _f32)
        return carry
    lax.fori_loop(0, NP // L, zero, 0)

    pltpu.sync_copy(dst_hbm.at[pl.ds(w * NCHUNK, NCHUNK), :], dstbuf)
    ones = jnp.ones((L,), _f32)

    def count(r, carry):
        for j in range(CK // L):
            idx = dstbuf[r, pl.ds(j * L, L)]
            plsc.addupdate_scatter(cntbuf, [idx], ones)
        return carry
    lax.fori_loop(0, NCHUNK, count, 0)

    pltpu.sync_copy(cntbuf, partials.at[s])
    plsc.subcore_barrier()

    cols = NP // NS
    col0 = s * cols
    pltpu.sync_copy(partials.at[:, pl.ds(col0, cols)], redbuf)

    def red_col(j, carry):
        def red_row(t, acc):
            return acc + redbuf[t, pl.ds(j * L, L)]
        acc = lax.fori_loop(1, NS, red_row, redbuf[0, pl.ds(j * L, L)])
        sumbuf[pl.ds(j * L, L)] = acc
        return carry
    lax.fori_loop(0, cols // L, red_col, 0)

    pltpu.sync_copy(sumbuf, cnt_out.at[c, pl.ds(col0, cols)])


_deg_kernel = functools.partial(
    pl.kernel, _deg_body,
    out_type=jax.ShapeDtypeStruct((NC, NP), _f32),
    mesh=_mesh,
    compiler_params=pltpu.CompilerParams(needs_layout_passes=False),
    scratch_types=[
        pltpu.VMEM((NCHUNK, CK), jnp.int32),
        pltpu.VMEM((NP,), _f32),
        pltpu.VMEM_SHARED((NS, NP), _f32),
        pltpu.VMEM((NS, NP // NS), _f32),
        pltpu.VMEM((NP // NS,), _f32),
    ])()


def _agg_body(y_hbm, src_hbm, dst_hbm, parts_out, sbuf, dbuf, rbs, acc,
              sgs, sss):
    c = lax.axis_index("c")
    s = lax.axis_index("s")
    w = _wid(c, s)

    def zero_row(i, carry):
        for j in range(D // L):
            rbs[0][i, pl.ds(j * L, L)] = jnp.zeros((L,), _f32)
        return carry
    lax.fori_loop(0, CK, zero_row, 0)
    rpt = NP // NS
    for k in range(rpt // CK):
        pltpu.sync_copy(rbs[0], acc.at[pl.ds(s * rpt + k * CK, CK), :])
    plsc.subcore_barrier()

    def gather_start(g, b3):
        pltpu.async_copy(y_hbm.at[sbuf.at[g]], rbs[b3], sgs[b3])

    def gather_wait(g, b3):
        pltpu.make_async_copy(y_hbm.at[sbuf.at[g]], rbs[b3],
                              sgs[b3]).wait()

    def scatter_start(g, b3):
        pltpu.async_copy(rbs[b3], acc.at[dbuf.at[g]], sss[b3], add=True)

    def scatter_wait(g, b3):
        pltpu.make_async_copy(rbs[b3], acc.at[dbuf.at[g]],
                              sss[b3]).wait()

    PH = NCHUNK // 2
    for phase in range(2):
        pltpu.sync_copy(
            src_hbm.at[pl.ds(w * NCHUNK + phase * PH, PH), :], sbuf)
        pltpu.sync_copy(
            dst_hbm.at[pl.ds(w * NCHUNK + phase * PH, PH), :], dbuf)
        gather_start(0, 0)
        gather_start(1, 1)

        def sweep(t, carry):
            for u in range(3):
                g = 3 * t + u
                b3 = u

                @pl.when(g < PH)
                def _():
                    gather_wait(g, b3)
                    scatter_start(g, b3)

                @pl.when(jnp.logical_and(g >= 1, g + 2 < PH))
                def _():
                    scatter_wait(g - 1, (u + 2) % 3)

                @pl.when(g + 2 < PH)
                def _():
                    gather_start(g + 2, (u + 2) % 3)
            return carry
        lax.fori_loop(0, (PH + 2) // 3, sweep, 0)
        for g in (PH - 3, PH - 2, PH - 1):
            scatter_wait(g, g % 3)
    plsc.subcore_barrier()

    for k in range(rpt // CK):
        r0 = s * rpt + k * CK
        rb = rbs[k % 3]
        pltpu.sync_copy(acc.at[pl.ds(r0, CK), :], rb)
        pltpu.sync_copy(rb, parts_out.at[c, pl.ds(r0, CK), :])


_agg_kernel = functools.partial(
    pl.kernel, _agg_body,
    out_type=jax.ShapeDtypeStruct((NC, NP, D), _f32),
    mesh=_mesh,
    compiler_params=pltpu.CompilerParams(needs_layout_passes=False),
    scratch_types=[
        pltpu.VMEM((NCHUNK // 2, CK), jnp.int32),
        pltpu.VMEM((NCHUNK // 2, CK), jnp.int32),
        tuple(pltpu.VMEM((CK, D), _f32) for _ in range(3)),
        pltpu.VMEM_SHARED((NP, D), _f32),
        tuple(pltpu.SemaphoreType.DMA for _ in range(3)),
        tuple(pltpu.SemaphoreType.DMA for _ in range(3)),
    ])()


def _pool_body(h_hbm, batch_hbm, pooled_out, hbuf, bvmem, accv,
               partials, redbuf, outbuf):
    c = lax.axis_index("c")
    s = lax.axis_index("s")
    w = _wid(c, s)
    r0 = w * RPW

    neg = jnp.full((L,), -jnp.inf, _f32)

    def init(i, carry):
        accv[pl.ds(i * L, L)] = neg
        return carry
    lax.fori_loop(0, (NG + 1) * D // L, init, 0)

    pltpu.sync_copy(h_hbm.at[pl.ds(r0, RPW), :], hbuf)
    pltpu.sync_copy(batch_hbm.at[pl.ds(r0, RPW)], bvmem)

    lanes = lax.iota(jnp.int32, L)

    def row(r, carry):
        b_prev, run = carry
        b_r = plsc.load_gather(bvmem, [jnp.broadcast_to(r, (L,))])
        same = b_r == b_prev
        base = b_r * D
        new_run = []
        for j in range(D // L):
            v = hbuf[r, pl.ds(j * L, L)]
            m = jnp.where(same, jnp.maximum(run[j], v), v)
            plsc.store_scatter(accv, [base + (j * L) + lanes], m)
            new_run.append(m)
        return b_r, tuple(new_run)

    init_carry = (jnp.full((L,), -1, jnp.int32),
                  tuple(neg for _ in range(D // L)))
    lax.fori_loop(0, RPW, row, init_carry)

    pltpu.sync_copy(accv.at[pl.ds(0, NG * D)], partials.at[s])
    plsc.subcore_barrier()

    cols = NG * D // NS
    col0 = s * cols
    pltpu.sync_copy(partials.at[:, pl.ds(col0, cols)], redbuf)

    def red_col(j, carry):
        def red_row(t, acc):
            return jnp.maximum(acc, redbuf[t, pl.ds(j * L, L)])
        acc = lax.fori_loop(1, NS, red_row, redbuf[0, pl.ds(j * L, L)])
        outbuf[pl.ds(j * L, L)] = acc
        return carry
    lax.fori_loop(0, cols // L, red_col, 0)

    pltpu.sync_copy(outbuf, pooled_out.at[c, pl.ds(col0, cols)])


_pool_kernel = functools.partial(
    pl.kernel, _pool_body,
    out_type=jax.ShapeDtypeStruct((NC, NG * D), _f32),
    mesh=_mesh,
    compiler_params=pltpu.CompilerParams(needs_layout_passes=False),
    scratch_types=[
        pltpu.VMEM((RPW, D), _f32),
        pltpu.VMEM((RPW,), jnp.int32),
        pltpu.VMEM(((NG + 1) * D,), _f32),
        pltpu.VMEM_SHARED((NS, NG * D), _f32),
        pltpu.VMEM((NS, NG * D // NS), _f32),
        pltpu.VMEM((NG * D // NS,), _f32),
    ])()


_GRID = NP // RPW

_row_spec = pl.BlockSpec((RPW, D), lambda i: (i, 0))
_cnt_spec = pl.BlockSpec((RPW, 1), lambda i: (i, 0))
_full_spec = pl.BlockSpec((D, D), lambda i: (0, 0))
_bias_spec = pl.BlockSpec((1, D), lambda i: (0, 0))


def _scale_mm_body(c0_ref, c1_ref, x_ref, w_ref, o_ref):
    dinv = lax.rsqrt(1.0 + c0_ref[...] + c1_ref[...])
    o_ref[...] = jnp.dot(x_ref[...], w_ref[...],
                         preferred_element_type=_f32) * dinv


def _tc_scale_mm(c0, c1, x, w):
    return pl.pallas_call(
        _scale_mm_body,
        grid=(N // 400,),
        in_specs=[pl.BlockSpec((400, 1), lambda i: (i, 0)),
                  pl.BlockSpec((400, 1), lambda i: (i, 0)),
                  pl.BlockSpec((400, D), lambda i: (i, 0)),
                  _full_spec],
        out_specs=pl.BlockSpec((400, D), lambda i: (i, 0)),
        out_shape=jax.ShapeDtypeStruct((NP, D), _f32),
    )(c0, c1, x, w)


def _combine_mm_body(c0_ref, c1_ref, s0_ref, s1_ref, y_ref, b_ref, w_ref,
                     o_ref):
    dinv = lax.rsqrt(1.0 + c0_ref[...] + c1_ref[...])
    h = jnp.maximum(
        dinv * (s0_ref[...] + s1_ref[...] + y_ref[...]) + b_ref[...], 0.0)
    o_ref[...] = jnp.dot(h, w_ref[...], preferred_element_type=_f32) * dinv


def _tc_combine_mm(c0, c1, s0, s1, y, b, w):
    return pl.pallas_call(
        _combine_mm_body,
        grid=(_GRID,),
        in_specs=[_cnt_spec, _cnt_spec, _row_spec, _row_spec, _row_spec,
                  _bias_spec, _full_spec],
        out_specs=_row_spec,
        out_shape=jax.ShapeDtypeStruct((NP, D), _f32),
    )(c0, c1, s0, s1, y, b, w)


def _combine_body(c0_ref, c1_ref, s0_ref, s1_ref, y_ref, b_ref, o_ref):
    dinv = lax.rsqrt(1.0 + c0_ref[...] + c1_ref[...])
    o_ref[...] = jnp.maximum(
        dinv * (s0_ref[...] + s1_ref[...] + y_ref[...]) + b_ref[...], 0.0)


def _tc_combine(c0, c1, s0, s1, y, b):
    return pl.pallas_call(
        _combine_body,
        grid=(_GRID,),
        in_specs=[_cnt_spec, _cnt_spec, _row_spec, _row_spec, _row_spec,
                  _bias_spec],
        out_specs=_row_spec,
        out_shape=jax.ShapeDtypeStruct((NP, D), _f32),
    )(c0, c1, s0, s1, y, b)


def _head_body(p_ref, fw_ref, fb_ref, lw_ref, lb_ref, o_ref):
    pooled = jnp.max(p_ref[...], axis=0)
    z = jnp.maximum(
        jnp.dot(pooled, fw_ref[...], preferred_element_type=_f32)
        + fb_ref[...], 0.0)
    o_ref[...] = jnp.dot(z, lw_ref[...], preferred_element_type=_f32) \
        + lb_ref[...]


def _tc_head(pooled_parts, fc1_W, fc1_b, lin_W, lin_b):
    ncls = lin_W.shape[1]
    return pl.pallas_call(
        _head_body,
        in_specs=[
            pl.BlockSpec((NC, NG, D), lambda: (0, 0, 0)),
            pl.BlockSpec((D, D), lambda: (0, 0)),
            pl.BlockSpec((1, D), lambda: (0, 0)),
            pl.BlockSpec((D, ncls), lambda: (0, 0)),
            pl.BlockSpec((1, ncls), lambda: (0, 0)),
        ],
        out_specs=pl.BlockSpec((NG, ncls), lambda: (0, 0)),
        out_shape=jax.ShapeDtypeStruct((NG, ncls), _f32),
    )(pooled_parts, fc1_W, fc1_b, lin_W, lin_b)


@jax.jit
def kernel(x, edge_index, batch, W1, b1, W2, b2, W3, b3,
           fc1_W, fc1_b, lin_W, lin_b):
    spread = N + jnp.arange(EP - E, dtype=edge_index.dtype) % (NP - N)
    srcp = jnp.concatenate([edge_index[0], spread]).reshape(EP // CK, CK)
    dstp = jnp.concatenate([edge_index[1], spread]).reshape(EP // CK, CK)
    batchp = jnp.concatenate(
        [batch, jnp.full((NP - N,), NG, batch.dtype)], axis=0)

    cnt = _deg_kernel(dstp)
    c0 = cnt[0].reshape(NP, 1)
    c1 = cnt[1].reshape(NP, 1)

    y1 = _tc_scale_mm(c0, c1, x, W1)
    s1 = _agg_kernel(y1, srcp, dstp)
    y2 = _tc_combine_mm(c0, c1, s1[0], s1[1], y1, b1.reshape(1, D), W2)
    s2 = _agg_kernel(y2, srcp, dstp)
    y3 = _tc_combine_mm(c0, c1, s2[0], s2[1], y2, b2.reshape(1, D), W3)
    s3 = _agg_kernel(y3, srcp, dstp)
    h3 = _tc_combine(c0, c1, s3[0], s3[1], y3, b3.reshape(1, D))

    pooled = _pool_kernel(h3, batchp).reshape(NC, NG, D)
    return _tc_head(pooled, fc1_W, fc1_b.reshape(1, D),
                    lin_W, lin_b.reshape(1, lin_W.shape[1]))


# speedup vs baseline: 1.0591x; 1.0189x over previous
import functools
import jax
import jax.numpy as jnp
from jax import lax
from jax.experimental import pallas as pl
from jax.experimental.pallas import tpu as pltpu
from jax.experimental.pallas import tpu_sc as plsc

N = 10000
E = 320000
D = 128
NG = 64
NC, NS, L = 2, 16, 16
NW = NC * NS
NP = 10240
RPW = NP // NW
CK = 80
NCHUNK = 128
EP = NW * NCHUNK * CK
DUMMY_SRC = N + 200
DUMMY_DST = NP - 1
EPT = EP // NW
ZR = 160

_mesh = plsc.VectorSubcoreMesh(
    core_axis_name="c", subcore_axis_name="s", num_cores=NC, num_subcores=NS)

_f32 = jnp.float32


def _wid(c, s):
    return c * NS + s


def _deg_body(dst_hbm, cnt_out, dstbuf, cntbuf, partials, redbuf, sumbuf):
    c = lax.axis_index("c")
    s = lax.axis_index("s")
    w = _wid(c, s)

    def zero(i, carry):
        cntbuf[pl.ds(i * L, L)] = jnp.zeros((L,), _f32)
        return carry
    lax.fori_loop(0, NP // L, zero, 0)

    pltpu.sync_copy(dst_hbm.at[pl.ds(w * NCHUNK, NCHUNK), :], dstbuf)
    ones = jnp.ones((L,), _f32)

    def count(r, carry):
        for j in range(CK // L):
            idx = dstbuf[r, pl.ds(j * L, L)]
            plsc.addupdate_scatter(cntbuf, [idx], ones)
        return carry
    lax.fori_loop(0, NCHUNK, count, 0)

    pltpu.sync_copy(cntbuf, partials.at[s])
    plsc.subcore_barrier()

    cols = NP // NS
    col0 = s * cols
    pltpu.sync_copy(partials.at[:, pl.ds(col0, cols)], redbuf)

    def red_col(j, carry):
        def red_row(t, acc):
            return acc + redbuf[t, pl.ds(j * L, L)]
        acc = lax.fori_loop(1, NS, red_row, redbuf[0, pl.ds(j * L, L)])
        sumbuf[pl.ds(j * L, L)] = acc
        return carry
    lax.fori_loop(0, cols // L, red_col, 0)

    pltpu.sync_copy(sumbuf, cnt_out.at[c, pl.ds(col0, cols)])


_deg_kernel = functools.partial(
    pl.kernel, _deg_body,
    out_type=jax.ShapeDtypeStruct((NC, NP), _f32),
    mesh=_mesh,
    compiler_params=pltpu.CompilerParams(needs_layout_passes=False),
    scratch_types=[
        pltpu.VMEM((NCHUNK, CK), jnp.int32),
        pltpu.VMEM((NP,), _f32),
        pltpu.VMEM_SHARED((NS, NP), _f32),
        pltpu.VMEM((NS, NP // NS), _f32),
        pltpu.VMEM((NP // NS,), _f32),
    ])()


def _agg_body(y_hbm, src_hbm, dst_hbm, parts_out, ibs, rbs, acc,
              sis, sgs, sss):
    c = lax.axis_index("c")
    s = lax.axis_index("s")
    w = _wid(c, s)

    def zero_row(i, carry):
        for j in range(D // L):
            rbs[0][i, pl.ds(j * L, L)] = jnp.zeros((L,), _f32)
        return carry
    lax.fori_loop(0, CK, zero_row, 0)
    rpt = NP // NS
    for k in range(rpt // CK):
        pltpu.sync_copy(rbs[0], acc.at[pl.ds(s * rpt + k * CK, CK), :])
    plsc.subcore_barrier()

    def idx_start(g, b6):
        pltpu.async_copy(src_hbm.at[w * NCHUNK + g], ibs[b6].at[0],
                         sis[b6])
        pltpu.async_copy(dst_hbm.at[w * NCHUNK + g], ibs[b6].at[1],
                         sis[b6])

    def idx_wait(g, b6):
        pltpu.make_async_copy(src_hbm.at[w * NCHUNK + g], ibs[b6].at[0],
                              sis[b6]).wait()
        pltpu.make_async_copy(dst_hbm.at[w * NCHUNK + g], ibs[b6].at[1],
                              sis[b6]).wait()

    def gather_start(b6, b3):
        pltpu.async_copy(y_hbm.at[ibs[b6].at[0]], rbs[b3], sgs[b3])

    def gather_wait(b6, b3):
        pltpu.make_async_copy(y_hbm.at[ibs[b6].at[0]], rbs[b3],
                              sgs[b3]).wait()

    def scatter_start(b6, b3):
        pltpu.async_copy(rbs[b3], acc.at[ibs[b6].at[1]], sss[b3], add=True)

    def scatter_wait(b6, b3):
        pltpu.make_async_copy(rbs[b3], acc.at[ibs[b6].at[1]],
                              sss[b3]).wait()

    idx_start(0, 0)
    idx_start(1, 1)
    idx_start(2, 2)
    idx_wait(0, 0)
    gather_start(0, 0)
    idx_wait(1, 1)
    gather_start(1, 1)

    def sweep(t, carry):
        for u in range(6):
            g = 6 * t + u
            b3 = u % 3
            b6 = u

            @pl.when(g < NCHUNK)
            def _():
                gather_wait(b6, b3)
                scatter_start(b6, b3)

            @pl.when(jnp.logical_and(g >= 1, g + 2 < NCHUNK))
            def _():
                scatter_wait((u + 2) % 6, (u + 2) % 3)

            @pl.when(g + 2 < NCHUNK)
            def _():
                idx_wait(g + 2, (u + 2) % 6)
                gather_start((u + 2) % 6, (u + 2) % 3)

            @pl.when(g + 3 < NCHUNK)
            def _():
                idx_start(g + 3, (u + 3) % 6)
        return carry
    lax.fori_loop(0, (NCHUNK + 5) // 6, sweep, 0)
    for g in (NCHUNK - 3, NCHUNK - 2, NCHUNK - 1):
        scatter_wait(g % 6, g % 3)
    plsc.subcore_barrier()

    for k in range(rpt // CK):
        r0 = s * rpt + k * CK
        rb = rbs[k % 3]
        pltpu.sync_copy(acc.at[pl.ds(r0, CK), :], rb)
        pltpu.sync_copy(rb, parts_out.at[c, pl.ds(r0, CK), :])


_agg_kernel = functools.partial(
    pl.kernel, _agg_body,
    out_type=jax.ShapeDtypeStruct((NC, NP, D), _f32),
    mesh=_mesh,
    compiler_params=pltpu.CompilerParams(needs_layout_passes=False),
    scratch_types=[
        tuple(pltpu.VMEM((2, CK), jnp.int32) for _ in range(6)),
        tuple(pltpu.VMEM((CK, D), _f32) for _ in range(3)),
        pltpu.VMEM_SHARED((NP, D), _f32),
        tuple(pltpu.SemaphoreType.DMA for _ in range(6)),
        tuple(pltpu.SemaphoreType.DMA for _ in range(3)),
        tuple(pltpu.SemaphoreType.DMA for _ in range(3)),
    ])()


def _pool_body(h_hbm, batch_hbm, pooled_out, hbuf, bvmem, accv,
               partials, redbuf, outbuf):
    c = lax.axis_index("c")
    s = lax.axis_index("s")
    w = _wid(c, s)
    r0 = w * RPW

    neg = jnp.full((L,), -jnp.inf, _f32)

    def init(i, carry):
        accv[pl.ds(i * L, L)] = neg
        return carry
    lax.fori_loop(0, (NG + 1) * D // L, init, 0)

    pltpu.sync_copy(h_hbm.at[pl.ds(r0, RPW), :], hbuf)
    pltpu.sync_copy(batch_hbm.at[pl.ds(r0, RPW)], bvmem)

    lanes = lax.iota(jnp.int32, L)

    def row(r, carry):
        b_prev, run = carry
        b_r = plsc.load_gather(bvmem, [jnp.broadcast_to(r, (L,))])
        same = b_r == b_prev
        base = b_r * D
        new_run = []
        for j in range(D // L):
            v = hbuf[r, pl.ds(j * L, L)]
            m = jnp.where(same, jnp.maximum(run[j], v), v)
            plsc.store_scatter(accv, [base + (j * L) + lanes], m)
            new_run.append(m)
        return b_r, tuple(new_run)

    init_carry = (jnp.full((L,), -1, jnp.int32),
                  tuple(neg for _ in range(D // L)))
    lax.fori_loop(0, RPW, row, init_carry)

    pltpu.sync_copy(accv.at[pl.ds(0, NG * D)], partials.at[s])
    plsc.subcore_barrier()

    cols = NG * D // NS
    col0 = s * cols
    pltpu.sync_copy(partials.at[:, pl.ds(col0, cols)], redbuf)

    def red_col(j, carry):
        def red_row(t, acc):
            return jnp.maximum(acc, redbuf[t, pl.ds(j * L, L)])
        acc = lax.fori_loop(1, NS, red_row, redbuf[0, pl.ds(j * L, L)])
        outbuf[pl.ds(j * L, L)] = acc
        return carry
    lax.fori_loop(0, cols // L, red_col, 0)

    pltpu.sync_copy(outbuf, pooled_out.at[c, pl.ds(col0, cols)])


_pool_kernel = functools.partial(
    pl.kernel, _pool_body,
    out_type=jax.ShapeDtypeStruct((NC, NG * D), _f32),
    mesh=_mesh,
    compiler_params=pltpu.CompilerParams(needs_layout_passes=False),
    scratch_types=[
        pltpu.VMEM((RPW, D), _f32),
        pltpu.VMEM((RPW,), jnp.int32),
        pltpu.VMEM(((NG + 1) * D,), _f32),
        pltpu.VMEM_SHARED((NS, NG * D), _f32),
        pltpu.VMEM((NS, NG * D // NS), _f32),
        pltpu.VMEM((NG * D // NS,), _f32),
    ])()


_GRID = NP // RPW

_row_spec = pl.BlockSpec((RPW, D), lambda i: (i, 0))
_cnt_spec = pl.BlockSpec((RPW, 1), lambda i: (i, 0))
_full_spec = pl.BlockSpec((D, D), lambda i: (0, 0))
_bias_spec = pl.BlockSpec((1, D), lambda i: (0, 0))


def _scale_mm_body(c0_ref, c1_ref, x_ref, w_ref, o_ref):
    dinv = lax.rsqrt(1.0 + c0_ref[...] + c1_ref[...])
    o_ref[...] = jnp.dot(x_ref[...], w_ref[...],
                         preferred_element_type=_f32) * dinv


def _tc_scale_mm(c0, c1, x, w):
    return pl.pallas_call(
        _scale_mm_body,
        grid=(N // 400,),
        in_specs=[pl.BlockSpec((400, 1), lambda i: (i, 0)),
                  pl.BlockSpec((400, 1), lambda i: (i, 0)),
                  pl.BlockSpec((400, D), lambda i: (i, 0)),
                  _full_spec],
        out_specs=pl.BlockSpec((400, D), lambda i: (i, 0)),
        out_shape=jax.ShapeDtypeStruct((NP, D), _f32),
    )(c0, c1, x, w)


def _combine_mm_body(c0_ref, c1_ref, s0_ref, s1_ref, y_ref, b_ref, w_ref,
                     o_ref):
    dinv = lax.rsqrt(1.0 + c0_ref[...] + c1_ref[...])
    h = jnp.maximum(
        dinv * (s0_ref[...] + s1_ref[...] + y_ref[...]) + b_ref[...], 0.0)
    o_ref[...] = jnp.dot(h, w_ref[...], preferred_element_type=_f32) * dinv


def _tc_combine_mm(c0, c1, s0, s1, y, b, w):
    return pl.pallas_call(
        _combine_mm_body,
        grid=(_GRID,),
        in_specs=[_cnt_spec, _cnt_spec, _row_spec, _row_spec, _row_spec,
                  _bias_spec, _full_spec],
        out_specs=_row_spec,
        out_shape=jax.ShapeDtypeStruct((NP, D), _f32),
    )(c0, c1, s0, s1, y, b, w)


def _combine_body(c0_ref, c1_ref, s0_ref, s1_ref, y_ref, b_ref, o_ref):
    dinv = lax.rsqrt(1.0 + c0_ref[...] + c1_ref[...])
    o_ref[...] = jnp.maximum(
        dinv * (s0_ref[...] + s1_ref[...] + y_ref[...]) + b_ref[...], 0.0)


def _tc_combine(c0, c1, s0, s1, y, b):
    return pl.pallas_call(
        _combine_body,
        grid=(_GRID,),
        in_specs=[_cnt_spec, _cnt_spec, _row_spec, _row_spec, _row_spec,
                  _bias_spec],
        out_specs=_row_spec,
        out_shape=jax.ShapeDtypeStruct((NP, D), _f32),
    )(c0, c1, s0, s1, y, b)


def _head_body(p_ref, fw_ref, fb_ref, lw_ref, lb_ref, o_ref):
    pooled = jnp.max(p_ref[...], axis=0)
    z = jnp.maximum(
        jnp.dot(pooled, fw_ref[...], preferred_element_type=_f32)
        + fb_ref[...], 0.0)
    o_ref[...] = jnp.dot(z, lw_ref[...], preferred_element_type=_f32) \
        + lb_ref[...]


def _tc_head(pooled_parts, fc1_W, fc1_b, lin_W, lin_b):
    ncls = lin_W.shape[1]
    return pl.pallas_call(
        _head_body,
        in_specs=[
            pl.BlockSpec((NC, NG, D), lambda: (0, 0, 0)),
            pl.BlockSpec((D, D), lambda: (0, 0)),
            pl.BlockSpec((1, D), lambda: (0, 0)),
            pl.BlockSpec((D, ncls), lambda: (0, 0)),
            pl.BlockSpec((1, ncls), lambda: (0, 0)),
        ],
        out_specs=pl.BlockSpec((NG, ncls), lambda: (0, 0)),
        out_shape=jax.ShapeDtypeStruct((NG, ncls), _f32),
    )(pooled_parts, fc1_W, fc1_b, lin_W, lin_b)


@jax.jit
def kernel(x, edge_index, batch, W1, b1, W2, b2, W3, b3,
           fc1_W, fc1_b, lin_W, lin_b):
    spread = N + jnp.arange(EP - E, dtype=edge_index.dtype) % (NP - N)
    srcp = jnp.concatenate([edge_index[0], spread]).reshape(EP // CK, CK)
    dstp = jnp.concatenate([edge_index[1], spread]).reshape(EP // CK, CK)
    batchp = jnp.concatenate(
        [batch, jnp.full((NP - N,), NG, batch.dtype)], axis=0)

    cnt = _deg_kernel(dstp)
    c0 = cnt[0].reshape(NP, 1)
    c1 = cnt[1].reshape(NP, 1)

    y1 = _tc_scale_mm(c0, c1, x, W1)
    s1 = _agg_kernel(y1, srcp, dstp)
    y2 = _tc_combine_mm(c0, c1, s1[0], s1[1], y1, b1.reshape(1, D), W2)
    s2 = _agg_kernel(y2, srcp, dstp)
    y3 = _tc_combine_mm(c0, c1, s2[0], s2[1], y2, b2.reshape(1, D), W3)
    s3 = _agg_kernel(y3, srcp, dstp)
    h3 = _tc_combine(c0, c1, s3[0], s3[1], y3, b3.reshape(1, D))

    pooled = _pool_kernel(h3, batchp).reshape(NC, NG, D)
    return _tc_head(pooled, fc1_W, fc1_b.reshape(1, D),
                    lin_W, lin_b.reshape(1, lin_W.shape[1]))


# speedup vs baseline: 1.1058x; 1.0441x over previous
import functools
import jax
import jax.numpy as jnp
from jax import lax
from jax.experimental import pallas as pl
from jax.experimental.pallas import tpu as pltpu
from jax.experimental.pallas import tpu_sc as plsc

N = 10000
E = 320000
D = 128
NG = 64
NC, NS, L = 2, 16, 16
NW = NC * NS
NP = 10240
RPW = NP // NW
CK = 80
NCHUNK = 128
EP = NW * NCHUNK * CK
DUMMY_SRC = N + 200
DUMMY_DST = NP - 1
EPT = EP // NW
ZR = 160

_mesh = plsc.VectorSubcoreMesh(
    core_axis_name="c", subcore_axis_name="s", num_cores=NC, num_subcores=NS)

_f32 = jnp.float32


def _wid(c, s):
    return c * NS + s


def _deg_body(dst_hbm, cnt_out, dstbuf, cntbuf, partials, redbuf, sumbuf):
    c = lax.axis_index("c")
    s = lax.axis_index("s")
    w = _wid(c, s)

    def zero(i, carry):
        cntbuf[pl.ds(i * L, L)] = jnp.zeros((L,), _f32)
        return carry
    lax.fori_loop(0, NP // L, zero, 0)

    pltpu.sync_copy(dst_hbm.at[pl.ds(w * NCHUNK, NCHUNK), :], dstbuf)
    ones = jnp.ones((L,), _f32)

    def count(r, carry):
        for j in range(CK // L):
            idx = dstbuf[r, pl.ds(j * L, L)]
            plsc.addupdate_scatter(cntbuf, [idx], ones)
        return carry
    lax.fori_loop(0, NCHUNK, count, 0)

    pltpu.sync_copy(cntbuf, partials.at[s])
    plsc.subcore_barrier()

    cols = NP // NS
    col0 = s * cols
    pltpu.sync_copy(partials.at[:, pl.ds(col0, cols)], redbuf)

    def red_col(j, carry):
        def red_row(t, acc):
            return acc + redbuf[t, pl.ds(j * L, L)]
        acc = lax.fori_loop(1, NS, red_row, redbuf[0, pl.ds(j * L, L)])
        sumbuf[pl.ds(j * L, L)] = acc
        return carry
    lax.fori_loop(0, cols // L, red_col, 0)

    pltpu.sync_copy(sumbuf, cnt_out.at[c, pl.ds(col0, cols)])


_deg_kernel = functools.partial(
    pl.kernel, _deg_body,
    out_type=jax.ShapeDtypeStruct((NC, NP), _f32),
    mesh=_mesh,
    compiler_params=pltpu.CompilerParams(needs_layout_passes=False),
    scratch_types=[
        pltpu.VMEM((NCHUNK, CK), jnp.int32),
        pltpu.VMEM((NP,), _f32),
        pltpu.VMEM_SHARED((NS, NP), _f32),
        pltpu.VMEM((NS, NP // NS), _f32),
        pltpu.VMEM((NP // NS,), _f32),
    ])()


def _agg_body(y_hbm, src_hbm, dst_hbm, parts_out, ibs, rbs, acc,
              sis, sgs, sss):
    c = lax.axis_index("c")
    s = lax.axis_index("s")
    w = _wid(c, s)

    def zero_row(i, carry):
        for j in range(D // L):
            rbs[0][i, pl.ds(j * L, L)] = jnp.zeros((L,), _f32)
        return carry
    lax.fori_loop(0, CK, zero_row, 0)
    rpt = NP // NS
    for k in range(rpt // CK):
        pltpu.sync_copy(rbs[0], acc.at[pl.ds(s * rpt + k * CK, CK), :])
    plsc.subcore_barrier()

    def idx_start(g, b6):
        pltpu.async_copy(src_hbm.at[w * NCHUNK + g], ibs[b6].at[0],
                         sis[b6])
        pltpu.async_copy(dst_hbm.at[w * NCHUNK + g], ibs[b6].at[1],
                         sis[b6])

    def idx_wait(g, b6):
        pltpu.make_async_copy(src_hbm.at[w * NCHUNK + g], ibs[b6].at[0],
                              sis[b6]).wait()
        pltpu.make_async_copy(dst_hbm.at[w * NCHUNK + g], ibs[b6].at[1],
                              sis[b6]).wait()

    def gather_start(b6, b3):
        pltpu.async_copy(y_hbm.at[ibs[b6].at[0]], rbs[b3], sgs[b3])

    def gather_wait(b6, b3):
        pltpu.make_async_copy(y_hbm.at[ibs[b6].at[0]], rbs[b3],
                              sgs[b3]).wait()

    def scatter_start(b6, b3):
        pltpu.async_copy(rbs[b3], acc.at[ibs[b6].at[1]], sss[b3], add=True)

    def scatter_wait(b6, b3):
        pltpu.make_async_copy(rbs[b3], acc.at[ibs[b6].at[1]],
                              sss[b3]).wait()

    idx_start(0, 0)
    idx_start(1, 1)
    idx_start(2, 2)
    idx_wait(0, 0)
    gather_start(0, 0)
    idx_wait(1, 1)
    gather_start(1, 1)

    def sweep(t, carry):
        for u in range(6):
            g = 6 * t + u
            b3 = u % 3
            b6 = u

            @pl.when(g < NCHUNK)
            def _():
                gather_wait(b6, b3)
                scatter_start(b6, b3)

            @pl.when(jnp.logical_and(g >= 1, g + 2 < NCHUNK))
            def _():
                scatter_wait((u + 2) % 6, (u + 2) % 3)

            @pl.when(g + 2 < NCHUNK)
            def _():
                idx_wait(g + 2, (u + 2) % 6)
                gather_start((u + 2) % 6, (u + 2) % 3)

            @pl.when(g + 3 < NCHUNK)
            def _():
                idx_start(g + 3, (u + 3) % 6)
        return carry
    lax.fori_loop(0, (NCHUNK + 5) // 6, sweep, 0)
    for g in (NCHUNK - 3, NCHUNK - 2, NCHUNK - 1):
        scatter_wait(g % 6, g % 3)
    plsc.subcore_barrier()

    for k in range(rpt // CK):
        r0 = s * rpt + k * CK
        rb = rbs[k % 3]
        pltpu.sync_copy(acc.at[pl.ds(r0, CK), :], rb)
        pltpu.sync_copy(rb, parts_out.at[c, pl.ds(r0, CK), :])


_agg_kernel = functools.partial(
    pl.kernel, _agg_body,
    out_type=jax.ShapeDtypeStruct((NC, NP, D), _f32),
    mesh=_mesh,
    compiler_params=pltpu.CompilerParams(needs_layout_passes=False),
    scratch_types=[
        tuple(pltpu.VMEM((2, CK), jnp.int32) for _ in range(6)),
        tuple(pltpu.VMEM((CK, D), _f32) for _ in range(3)),
        pltpu.VMEM_SHARED((NP, D), _f32),
        tuple(pltpu.SemaphoreType.DMA for _ in range(6)),
        tuple(pltpu.SemaphoreType.DMA for _ in range(3)),
        tuple(pltpu.SemaphoreType.DMA for _ in range(3)),
    ])()


def _pool_body(h_hbm, batch_hbm, pooled_out, hbuf, bvmem, accv,
               partials, redbuf, outbuf):
    c = lax.axis_index("c")
    s = lax.axis_index("s")
    w = _wid(c, s)
    r0 = w * RPW

    neg = jnp.full((L,), -jnp.inf, _f32)

    def init(i, carry):
        accv[pl.ds(i * L, L)] = neg
        return carry
    lax.fori_loop(0, (NG + 1) * D // L, init, 0)

    pltpu.sync_copy(h_hbm.at[pl.ds(r0, RPW), :], hbuf)
    pltpu.sync_copy(batch_hbm.at[pl.ds(r0, RPW)], bvmem)

    lanes = lax.iota(jnp.int32, L)

    def row(r, carry):
        b_prev, run = carry
        b_r = plsc.load_gather(bvmem, [jnp.broadcast_to(r, (L,))])
        same = b_r == b_prev
        base = b_r * D
        new_run = []
        for j in range(D // L):
            v = hbuf[r, pl.ds(j * L, L)]
            m = jnp.where(same, jnp.maximum(run[j], v), v)
            plsc.store_scatter(accv, [base + (j * L) + lanes], m)
            new_run.append(m)
        return b_r, tuple(new_run)

    init_carry = (jnp.full((L,), -1, jnp.int32),
                  tuple(neg for _ in range(D // L)))
    lax.fori_loop(0, RPW, row, init_carry)

    pltpu.sync_copy(accv.at[pl.ds(0, NG * D)], partials.at[s])
    plsc.subcore_barrier()

    cols = NG * D // NS
    col0 = s * cols
    pltpu.sync_copy(partials.at[:, pl.ds(col0, cols)], redbuf)

    def red_col(j, carry):
        def red_row(t, acc):
            return jnp.maximum(acc, redbuf[t, pl.ds(j * L, L)])
        acc = lax.fori_loop(1, NS, red_row, redbuf[0, pl.ds(j * L, L)])
        outbuf[pl.ds(j * L, L)] = acc
        return carry
    lax.fori_loop(0, cols // L, red_col, 0)

    pltpu.sync_copy(outbuf, pooled_out.at[c, pl.ds(col0, cols)])


_pool_kernel = functools.partial(
    pl.kernel, _pool_body,
    out_type=jax.ShapeDtypeStruct((NC, NG * D), _f32),
    mesh=_mesh,
    compiler_params=pltpu.CompilerParams(needs_layout_passes=False),
    scratch_types=[
        pltpu.VMEM((RPW, D), _f32),
        pltpu.VMEM((RPW,), jnp.int32),
        pltpu.VMEM(((NG + 1) * D,), _f32),
        pltpu.VMEM_SHARED((NS, NG * D), _f32),
        pltpu.VMEM((NS, NG * D // NS), _f32),
        pltpu.VMEM((NG * D // NS,), _f32),
    ])()


_GRID = NP // RPW

_row_spec = pl.BlockSpec((RPW, D), lambda i: (i, 0))
_cnt_spec = pl.BlockSpec((RPW, 1), lambda i: (i, 0))
_full_spec = pl.BlockSpec((D, D), lambda i: (0, 0))
_bias_spec = pl.BlockSpec((1, D), lambda i: (0, 0))


def _scale_mm_body(c0_ref, c1_ref, x_ref, w_ref, o_ref):
    dinv = lax.rsqrt(1.0 + c0_ref[...] + c1_ref[...])
    o_ref[...] = jnp.dot(x_ref[...], w_ref[...],
                         preferred_element_type=_f32) * dinv


def _tc_scale_mm(c0, c1, x, w):
    return pl.pallas_call(
        _scale_mm_body,
        grid=(N // 400,),
        in_specs=[pl.BlockSpec((400, 1), lambda i: (i, 0)),
                  pl.BlockSpec((400, 1), lambda i: (i, 0)),
                  pl.BlockSpec((400, D), lambda i: (i, 0)),
                  _full_spec],
        out_specs=pl.BlockSpec((400, D), lambda i: (i, 0)),
        out_shape=jax.ShapeDtypeStruct((NP, D), _f32),
    )(c0, c1, x, w)


def _combine_mm_body(c0_ref, c1_ref, s0_ref, s1_ref, y_ref, b_ref, w_ref,
                     o_ref):
    dinv = lax.rsqrt(1.0 + c0_ref[...] + c1_ref[...])
    h = jnp.maximum(
        dinv * (s0_ref[0] + s1_ref[0] + y_ref[...]) + b_ref[...], 0.0)
    o_ref[...] = jnp.dot(h, w_ref[...], preferred_element_type=_f32) * dinv


_s0_spec = pl.BlockSpec((1, RPW, D), lambda i: (0, i, 0))
_s1_spec = pl.BlockSpec((1, RPW, D), lambda i: (1, i, 0))


def _tc_combine_mm(c0, c1, sp, y, b, w):
    return pl.pallas_call(
        _combine_mm_body,
        grid=(_GRID,),
        in_specs=[_cnt_spec, _cnt_spec, _s0_spec, _s1_spec, _row_spec,
                  _bias_spec, _full_spec],
        out_specs=_row_spec,
        out_shape=jax.ShapeDtypeStruct((NP, D), _f32),
    )(c0, c1, sp, sp, y, b, w)


def _combine_body(c0_ref, c1_ref, s0_ref, s1_ref, y_ref, b_ref, o_ref):
    dinv = lax.rsqrt(1.0 + c0_ref[...] + c1_ref[...])
    o_ref[...] = jnp.maximum(
        dinv * (s0_ref[0] + s1_ref[0] + y_ref[...]) + b_ref[...], 0.0)


def _tc_combine(c0, c1, sp, y, b):
    return pl.pallas_call(
        _combine_body,
        grid=(_GRID,),
        in_specs=[_cnt_spec, _cnt_spec, _s0_spec, _s1_spec, _row_spec,
                  _bias_spec],
        out_specs=_row_spec,
        out_shape=jax.ShapeDtypeStruct((NP, D), _f32),
    )(c0, c1, sp, sp, y, b)


def _head_body(p_ref, fw_ref, fb_ref, lw_ref, lb_ref, o_ref):
    pooled = jnp.max(p_ref[...], axis=0)
    z = jnp.maximum(
        jnp.dot(pooled, fw_ref[...], preferred_element_type=_f32)
        + fb_ref[...], 0.0)
    o_ref[...] = jnp.dot(z, lw_ref[...], preferred_element_type=_f32) \
        + lb_ref[...]


def _tc_head(pooled_parts, fc1_W, fc1_b, lin_W, lin_b):
    ncls = lin_W.shape[1]
    return pl.pallas_call(
        _head_body,
        in_specs=[
            pl.BlockSpec((NC, NG, D), lambda: (0, 0, 0)),
            pl.BlockSpec((D, D), lambda: (0, 0)),
            pl.BlockSpec((1, D), lambda: (0, 0)),
            pl.BlockSpec((D, ncls), lambda: (0, 0)),
            pl.BlockSpec((1, ncls), lambda: (0, 0)),
        ],
        out_specs=pl.BlockSpec((NG, ncls), lambda: (0, 0)),
        out_shape=jax.ShapeDtypeStruct((NG, ncls), _f32),
    )(pooled_parts, fc1_W, fc1_b, lin_W, lin_b)


@jax.jit
def kernel(x, edge_index, batch, W1, b1, W2, b2, W3, b3,
           fc1_W, fc1_b, lin_W, lin_b):
    spread = N + jnp.arange(EP - E, dtype=edge_index.dtype) % (NP - N)
    srcp = jnp.concatenate([edge_index[0], spread]).reshape(EP // CK, CK)
    dstp = jnp.concatenate([edge_index[1], spread]).reshape(EP // CK, CK)
    batchp = jnp.concatenate(
        [batch, jnp.full((NP - N,), NG, batch.dtype)], axis=0)

    cnt = _deg_kernel(dstp)
    c0 = cnt[0].reshape(NP, 1)
    c1 = cnt[1].reshape(NP, 1)

    y1 = _tc_scale_mm(c0, c1, x, W1)
    s1 = _agg_kernel(y1, srcp, dstp)
    y2 = _tc_combine_mm(c0, c1, s1, y1, b1.reshape(1, D), W2)
    s2 = _agg_kernel(y2, srcp, dstp)
    y3 = _tc_combine_mm(c0, c1, s2, y2, b2.reshape(1, D), W3)
    s3 = _agg_kernel(y3, srcp, dstp)
    h3 = _tc_combine(c0, c1, s3, y3, b3.reshape(1, D))

    pooled = _pool_kernel(h3, batchp).reshape(NC, NG, D)
    return _tc_head(pooled, fc1_W, fc1_b.reshape(1, D),
                    lin_W, lin_b.reshape(1, lin_W.shape[1]))
